# baseline (speedup 1.0000x reference)
import functools

import jax
import jax.numpy as jnp
from jax import lax
from jax.experimental import pallas as pl
from jax.experimental.pallas import tpu as pltpu
from jax.experimental.pallas import tpu_sc as plsc

_L = 16
_CH = 128
_G = _CH // _L


def _pers_from_lanes(x, y, z, cam):
    r00, r01, r02, r10, r11, r12, r20, r21, r22, c0, c1, c2 = cam
    sx = x - c0
    sy = y - c1
    sz = z - c2
    u0 = r00 * sx + r10 * sy + r20 * sz
    u1 = r01 * sx + r11 * sy + r21 * sz
    u2 = r02 * sx + r12 * sy + r22 * sz
    return u0 / u2, u1 / u2, u2


def _body(pk_hbm, emb_hbm, idx_hbm, locw_hbm, cam_hbm,
          oemb_hbm, ocol_hbm, odir_hbm, oconf_hbm, oloc_hbm,
          idx_v, pk_v, emb_v, oemb_v, ocol_v, odir_v, oconf_v,
          loc_v, oloc_v, cam_v, sem, osem):
    q_total = idx_hbm.shape[0]
    qloc_total = locw_hbm.shape[0] // 3
    nw = 32
    qw = q_total // nw
    qlw = qloc_total // nw
    wid = lax.axis_index("s") * 2 + lax.axis_index("c")

    pltpu.sync_copy(cam_hbm, cam_v)
    cam = tuple(cam_v[j, :] for j in range(12))
    iota = lax.iota(jnp.int32, _L)
    iota3 = iota * 3
    cols = [jnp.full((_L,), c, jnp.int32) for c in range(10)]

    def chunk(it, carry):
        base = wid * qw + it * _CH
        pltpu.sync_copy(idx_hbm.at[pl.ds(base, _CH)], idx_v)
        d0 = pltpu.async_copy(emb_hbm.at[idx_v], emb_v, sem)
        d1 = pltpu.async_copy(pk_hbm.at[idx_v], pk_v, sem)
        d0.wait()
        d1.wait()
        for g in range(_G):
            rows = iota + g * _L
            f38 = iota * 38 + g * (_L * 38)
            f3 = iota3 + g * (_L * 3)
            x = plsc.load_gather(pk_v, [rows, cols[0]])
            y = plsc.load_gather(pk_v, [rows, cols[1]])
            z = plsc.load_gather(pk_v, [rows, cols[2]])
            xp, yp, zc = _pers_from_lanes(x, y, z, cam)
            for c, val in ((0, x), (1, y), (2, z), (3, xp), (4, yp), (5, zc)):
                plsc.store_scatter(oemb_v, [f38 + c], val)
            for c in range(3):
                plsc.store_scatter(
                    ocol_v, [f3 + c], plsc.load_gather(pk_v, [rows, cols[3 + c]]))
            for c in range(3):
                plsc.store_scatter(
                    odir_v, [f3 + c], plsc.load_gather(pk_v, [rows, cols[6 + c]]))
            plsc.store_scatter(
                oconf_v, [rows], plsc.load_gather(pk_v, [rows, cols[9]]))
        for p in range(_CH):
            oemb_v[pl.ds(p * 38 + 6, 16)] = emb_v[p, pl.ds(0, 16)]
            oemb_v[pl.ds(p * 38 + 22, 16)] = emb_v[p, pl.ds(16, 16)]
        o = [
            pltpu.async_copy(oemb_v, oemb_hbm.at[pl.ds(base * 38, _CH * 38)],
                             osem),
            pltpu.async_copy(ocol_v, ocol_hbm.at[pl.ds(base * 3, _CH * 3)],
                             osem),
            pltpu.async_copy(odir_v, odir_hbm.at[pl.ds(base * 3, _CH * 3)],
                             osem),
            pltpu.async_copy(oconf_v, oconf_hbm.at[pl.ds(base, _CH)], osem),
        ]
        for oo in o:
            oo.wait()
        return carry

    lax.fori_loop(0, qw // _CH, chunk, 0)

    def loc_chunk(it, carry):
        base = wid * qlw + it * _CH
        pltpu.sync_copy(locw_hbm.at[pl.ds(base * 3, _CH * 3)], loc_v)
        for g in range(_G):
            f3 = iota3 + g * (_L * 3)
            x = plsc.load_gather(loc_v, [f3])
            y = plsc.load_gather(loc_v, [f3 + 1])
            z = plsc.load_gather(loc_v, [f3 + 2])
            xp, yp, zc = _pers_from_lanes(x, y, z, cam)
            plsc.store_scatter(oloc_v, [f3], xp)
            plsc.store_scatter(oloc_v, [f3 + 1], yp)
            plsc.store_scatter(oloc_v, [f3 + 2], zc)
        pltpu.sync_copy(oloc_v, oloc_hbm.at[pl.ds(base * 3, _CH * 3)])
        return carry

    lax.fori_loop(0, qlw // _CH, loc_chunk, 0)


def kernel(points_xyz, points_embeding, points_conf, points_dir, points_color,
           sample_pidx, sample_loc_w, cam_rot, cam_pos):
    n = points_xyz.shape[0]
    b, r, sr, k = sample_pidx.shape
    c = points_embeding.shape[-1]
    q = b * r * sr * k
    qloc = b * r * sr
    cf = c + 6

    packed = jnp.concatenate(
        [points_xyz, points_color[0], points_dir[0], points_conf[0],
         jnp.zeros((n, 6), jnp.float32)], axis=-1)
    emb2d = points_embeding.reshape(n, c)
    idx1 = sample_pidx.reshape(q)
    locw1 = sample_loc_w.reshape(3 * qloc)
    cam12 = jnp.concatenate(
        [cam_rot.reshape(9), cam_pos.reshape(3)]).astype(jnp.float32)
    cam_b = jnp.tile(cam12[:, None], (1, _L))

    mesh = plsc.VectorSubcoreMesh(
        core_axis_name="c", subcore_axis_name="s", num_cores=2, num_subcores=16)
    f32 = jnp.float32
    i32 = jnp.int32
    out_type = (
        jax.ShapeDtypeStruct((q * cf,), f32),
        jax.ShapeDtypeStruct((q * 3,), f32),
        jax.ShapeDtypeStruct((q * 3,), f32),
        jax.ShapeDtypeStruct((q,), f32),
        jax.ShapeDtypeStruct((qloc * 3,), f32),
    )
    scratch = [
        pltpu.VMEM((_CH,), i32),
        pltpu.VMEM((_CH, 16), f32),
        pltpu.VMEM((_CH, c), f32),
        pltpu.VMEM((_CH * cf,), f32),
        pltpu.VMEM((_CH * 3,), f32),
        pltpu.VMEM((_CH * 3,), f32),
        pltpu.VMEM((_CH,), f32),
        pltpu.VMEM((_CH * 3,), f32),
        pltpu.VMEM((_CH * 3,), f32),
        pltpu.VMEM((12, _L), f32),
        pltpu.SemaphoreType.DMA,
        pltpu.SemaphoreType.DMA,
    ]
    fn = pl.kernel(
        _body, out_type=out_type, mesh=mesh, scratch_types=scratch,
        compiler_params=pltpu.CompilerParams(
            needs_layout_passes=False, use_tc_tiling_on_sc=False))
    oemb, ocol, odir, oconf, oloc = fn(packed, emb2d, idx1, locw1, cam_b)

    return (oemb.reshape(b, r, sr, k, cf),
            ocol.reshape(b, r, sr, k, 3),
            odir.reshape(b, r, sr, k, 3),
            oconf.reshape(b, r, sr, k, 1),
            oloc.reshape(b, r, sr, 3))


# speedup vs baseline: 1.0002x; 1.0002x over previous
import functools

import jax
import jax.numpy as jnp
from jax import lax
from jax.experimental import pallas as pl
from jax.experimental.pallas import tpu as pltpu
from jax.experimental.pallas import tpu_sc as plsc

_L = 16
_CH = 128
_G = _CH // _L


def _pers_from_lanes(x, y, z, cam):
    r00, r01, r02, r10, r11, r12, r20, r21, r22, c0, c1, c2 = cam
    sx = x - c0
    sy = y - c1
    sz = z - c2
    u0 = r00 * sx + r10 * sy + r20 * sz
    u1 = r01 * sx + r11 * sy + r21 * sz
    u2 = r02 * sx + r12 * sy + r22 * sz
    return u0 / u2, u1 / u2, u2


def _body(pk_hbm, emb_hbm, idx_hbm, locw_hbm, cam_hbm,
          oemb_hbm, ocol_hbm, odir_hbm, oconf_hbm, oloc_hbm,
          idx8_v, idx_v, pk_v, emb_v, oemb_v, ocol_v, odir_v, oconf_v,
          loc_v, oloc_v, cam_v, sem, osem):
    q_total = idx_hbm.shape[0] * 8
    qloc_total = locw_hbm.shape[0]
    nw = 32
    qw = q_total // nw
    qlw = qloc_total // nw
    wid = lax.axis_index("s") * 2 + lax.axis_index("c")

    pltpu.sync_copy(cam_hbm, cam_v)
    cam = tuple(cam_v[j, :] for j in range(12))
    iota = lax.iota(jnp.int32, _L)
    iota3 = iota * 3
    cols = [jnp.full((_L,), c, jnp.int32) for c in range(10)]

    def chunk(it, carry):
        base = wid * qw + it * _CH
        pltpu.sync_copy(idx_hbm.at[pl.ds(base // 8, _CH // 8)], idx8_v)
        for g in range(_G):
            e = iota + g * _L
            idx_v[pl.ds(g * _L, _L)] = plsc.load_gather(
                idx8_v, [lax.shift_right_logical(e, 3), lax.bitwise_and(e, 7)])
        d0 = pltpu.async_copy(emb_hbm.at[idx_v], emb_v, sem)
        d1 = pltpu.async_copy(pk_hbm.at[idx_v], pk_v, sem)
        d0.wait()
        d1.wait()
        for g in range(_G):
            rows = iota + g * _L
            f38 = iota * 38 + g * (_L * 38)
            f3 = iota3 + g * (_L * 3)
            x = plsc.load_gather(pk_v, [rows, cols[0]])
            y = plsc.load_gather(pk_v, [rows, cols[1]])
            z = plsc.load_gather(pk_v, [rows, cols[2]])
            xp, yp, zc = _pers_from_lanes(x, y, z, cam)
            for c, val in ((0, x), (1, y), (2, z), (3, xp), (4, yp), (5, zc)):
                plsc.store_scatter(oemb_v, [f38 + c], val)
            for c in range(3):
                plsc.store_scatter(
                    ocol_v, [f3 + c], plsc.load_gather(pk_v, [rows, cols[3 + c]]))
            for c in range(3):
                plsc.store_scatter(
                    odir_v, [f3 + c], plsc.load_gather(pk_v, [rows, cols[6 + c]]))
            plsc.store_scatter(
                oconf_v, [rows], plsc.load_gather(pk_v, [rows, cols[9]]))
        for p in range(_CH):
            oemb_v[pl.ds(p * 38 + 6, 16)] = emb_v[p, pl.ds(0, 16)]
            oemb_v[pl.ds(p * 38 + 22, 16)] = emb_v[p, pl.ds(16, 16)]
        o = [
            pltpu.async_copy(oemb_v, oemb_hbm.at[pl.ds(base * 38, _CH * 38)],
                             osem),
            pltpu.async_copy(ocol_v, ocol_hbm.at[pl.ds(base * 3, _CH * 3)],
                             osem),
            pltpu.async_copy(odir_v, odir_hbm.at[pl.ds(base * 3, _CH * 3)],
                             osem),
            pltpu.async_copy(oconf_v, oconf_hbm.at[pl.ds(base, _CH)], osem),
        ]
        for oo in o:
            oo.wait()
        return carry

    lax.fori_loop(0, qw // _CH, chunk, 0)

    def loc_chunk(it, carry):
        base = wid * qlw + it * _CH
        pltpu.sync_copy(locw_hbm.at[pl.ds(base, _CH)], loc_v)
        for g in range(_G):
            rows = iota + g * _L
            x = plsc.load_gather(loc_v, [rows, cols[0]])
            y = plsc.load_gather(loc_v, [rows, cols[1]])
            z = plsc.load_gather(loc_v, [rows, cols[2]])
            xp, yp, zc = _pers_from_lanes(x, y, z, cam)
            plsc.store_scatter(oloc_v, [rows, cols[0]], xp)
            plsc.store_scatter(oloc_v, [rows, cols[1]], yp)
            plsc.store_scatter(oloc_v, [rows, cols[2]], zc)
        pltpu.sync_copy(oloc_v, oloc_hbm.at[pl.ds(base, _CH)])
        return carry

    lax.fori_loop(0, qlw // _CH, loc_chunk, 0)


def kernel(points_xyz, points_embeding, points_conf, points_dir, points_color,
           sample_pidx, sample_loc_w, cam_rot, cam_pos):
    n = points_xyz.shape[0]
    b, r, sr, k = sample_pidx.shape
    c = points_embeding.shape[-1]
    q = b * r * sr * k
    qloc = b * r * sr
    cf = c + 6

    packed = jnp.concatenate(
        [points_xyz, points_color[0], points_dir[0], points_conf[0],
         jnp.zeros((n, 6), jnp.float32)], axis=-1)
    emb2d = points_embeding.reshape(n, c)
    idx8 = sample_pidx.reshape(q // 8, 8)
    locw3 = sample_loc_w.reshape(qloc, 3)
    cam12 = jnp.concatenate(
        [cam_rot.reshape(9), cam_pos.reshape(3)]).astype(jnp.float32)
    cam_b = jnp.tile(cam12[:, None], (1, _L))

    mesh = plsc.VectorSubcoreMesh(
        core_axis_name="c", subcore_axis_name="s", num_cores=2, num_subcores=16)
    f32 = jnp.float32
    i32 = jnp.int32
    out_type = (
        jax.ShapeDtypeStruct((q * cf,), f32),
        jax.ShapeDtypeStruct((q * 3,), f32),
        jax.ShapeDtypeStruct((q * 3,), f32),
        jax.ShapeDtypeStruct((q,), f32),
        jax.ShapeDtypeStruct((qloc, 3), f32),
    )
    scratch = [
        pltpu.VMEM((_CH // 8, 8), i32),
        pltpu.VMEM((_CH,), i32),
        pltpu.VMEM((_CH, 16), f32),
        pltpu.VMEM((_CH, c), f32),
        pltpu.VMEM((_CH * cf,), f32),
        pltpu.VMEM((_CH * 3,), f32),
        pltpu.VMEM((_CH * 3,), f32),
        pltpu.VMEM((_CH,), f32),
        pltpu.VMEM((_CH, 3), f32),
        pltpu.VMEM((_CH, 3), f32),
        pltpu.VMEM((12, _L), f32),
        pltpu.SemaphoreType.DMA,
        pltpu.SemaphoreType.DMA,
    ]
    fn = pl.kernel(
        _body, out_type=out_type, mesh=mesh, scratch_types=scratch,
        compiler_params=pltpu.CompilerParams(
            needs_layout_passes=False, use_tc_tiling_on_sc=False))
    oemb, ocol, odir, oconf, oloc = fn(packed, emb2d, idx8, locw3, cam_b)

    return (oemb.reshape(b, r, sr, k, cf),
            ocol.reshape(b, r, sr, k, 3),
            odir.reshape(b, r, sr, k, 3),
            oconf.reshape(b, r, sr, k, 1),
            oloc.reshape(b, r, sr, 3))


# speedup vs baseline: 1.0862x; 1.0859x over previous
import functools

import jax
import jax.numpy as jnp
from jax import lax
from jax.experimental import pallas as pl
from jax.experimental.pallas import tpu as pltpu
from jax.experimental.pallas import tpu_sc as plsc

_L = 16
_CH = 128
_G = _CH // _L


def _pers_from_lanes(x, y, z, cam):
    r00, r01, r02, r10, r11, r12, r20, r21, r22, c0, c1, c2 = cam
    sx = x - c0
    sy = y - c1
    sz = z - c2
    u0 = r00 * sx + r10 * sy + r20 * sz
    u1 = r01 * sx + r11 * sy + r21 * sz
    u2 = r02 * sx + r12 * sy + r22 * sz
    return u0 / u2, u1 / u2, u2


def _body(pk_hbm, emb_hbm, idx_hbm, locw_hbm, cam_hbm,
          oemb_hbm, ocol_hbm, odir_hbm, oconf_hbm, oloc_hbm,
          idx8_v, idx_v, pk_v, emb_v, oemb_v, ocol_v, odir_v, oconf_v,
          loc_v, oloc_v, cam_v, sem, osem):
    q_total = idx_hbm.shape[0] * 8
    qloc_total = locw_hbm.shape[0]
    nw = 32
    qw = q_total // nw
    qlw = qloc_total // nw
    wid = lax.axis_index("s") * 2 + lax.axis_index("c")

    pltpu.sync_copy(cam_hbm, cam_v)
    cam = tuple(cam_v[j, :] for j in range(12))
    iota = lax.iota(jnp.int32, _L)
    iota3 = iota * 3
    cols = [jnp.full((_L,), c, jnp.int32) for c in range(10)]

    def chunk(it, carry):
        base = wid * qw + it * _CH
        pltpu.sync_copy(idx_hbm.at[pl.ds(base // 8, _CH // 8)], idx8_v)
        for g in range(_G):
            e = iota + g * _L
            idx_v[pl.ds(g * _L, _L)] = plsc.load_gather(
                idx8_v, [lax.shift_right_logical(e, 3), lax.bitwise_and(e, 7)])
        d0 = pltpu.async_copy(emb_hbm.at[idx_v], emb_v, sem)
        d1 = pltpu.async_copy(pk_hbm.at[idx_v], pk_v, sem)
        d0.wait()
        d1.wait()
        for g in range(_G):
            rows = iota + g * _L
            x = plsc.load_gather(pk_v, [rows, cols[0]])
            y = plsc.load_gather(pk_v, [rows, cols[1]])
            z = plsc.load_gather(pk_v, [rows, cols[2]])
            xp, yp, zc = _pers_from_lanes(x, y, z, cam)
            for c, val in ((0, x), (1, y), (2, z), (3, xp), (4, yp), (5, zc)):
                plsc.store_scatter(oemb_v, [rows, cols[c]], val)
            for c in range(3):
                plsc.store_scatter(
                    ocol_v, [rows, cols[c]],
                    plsc.load_gather(pk_v, [rows, cols[3 + c]]))
            for c in range(3):
                plsc.store_scatter(
                    odir_v, [rows, cols[c]],
                    plsc.load_gather(pk_v, [rows, cols[6 + c]]))
            plsc.store_scatter(
                oconf_v, [rows], plsc.load_gather(pk_v, [rows, cols[9]]))
        for p in range(_CH):
            oemb_v[p, pl.ds(6, 16)] = emb_v[p, pl.ds(0, 16)]
            oemb_v[p, pl.ds(22, 16)] = emb_v[p, pl.ds(16, 16)]
        o = [
            pltpu.async_copy(oemb_v, oemb_hbm.at[pl.ds(base, _CH)], osem),
            pltpu.async_copy(ocol_v, ocol_hbm.at[pl.ds(base, _CH)], osem),
            pltpu.async_copy(odir_v, odir_hbm.at[pl.ds(base, _CH)], osem),
            pltpu.async_copy(oconf_v, oconf_hbm.at[pl.ds(base, _CH)], osem),
        ]
        for oo in o:
            oo.wait()
        return carry

    lax.fori_loop(0, qw // _CH, chunk, 0)

    def loc_chunk(it, carry):
        base = wid * qlw + it * _CH
        pltpu.sync_copy(locw_hbm.at[pl.ds(base, _CH)], loc_v)
        for g in range(_G):
            rows = iota + g * _L
            x = plsc.load_gather(loc_v, [rows, cols[0]])
            y = plsc.load_gather(loc_v, [rows, cols[1]])
            z = plsc.load_gather(loc_v, [rows, cols[2]])
            xp, yp, zc = _pers_from_lanes(x, y, z, cam)
            plsc.store_scatter(oloc_v, [rows, cols[0]], xp)
            plsc.store_scatter(oloc_v, [rows, cols[1]], yp)
            plsc.store_scatter(oloc_v, [rows, cols[2]], zc)
        pltpu.sync_copy(oloc_v, oloc_hbm.at[pl.ds(base, _CH)])
        return carry

    lax.fori_loop(0, qlw // _CH, loc_chunk, 0)


def kernel(points_xyz, points_embeding, points_conf, points_dir, points_color,
           sample_pidx, sample_loc_w, cam_rot, cam_pos):
    n = points_xyz.shape[0]
    b, r, sr, k = sample_pidx.shape
    c = points_embeding.shape[-1]
    q = b * r * sr * k
    qloc = b * r * sr
    cf = c + 6

    packed = jnp.concatenate(
        [points_xyz, points_color[0], points_dir[0], points_conf[0],
         jnp.zeros((n, 6), jnp.float32)], axis=-1)
    emb2d = points_embeding.reshape(n, c)
    idx8 = sample_pidx.reshape(q // 8, 8)
    locw3 = sample_loc_w.reshape(qloc, 3)
    cam12 = jnp.concatenate(
        [cam_rot.reshape(9), cam_pos.reshape(3)]).astype(jnp.float32)
    cam_b = jnp.tile(cam12[:, None], (1, _L))

    mesh = plsc.VectorSubcoreMesh(
        core_axis_name="c", subcore_axis_name="s", num_cores=2, num_subcores=16)
    f32 = jnp.float32
    i32 = jnp.int32
    out_type = (
        jax.ShapeDtypeStruct((q, cf), f32),
        jax.ShapeDtypeStruct((q, 3), f32),
        jax.ShapeDtypeStruct((q, 3), f32),
        jax.ShapeDtypeStruct((q,), f32),
        jax.ShapeDtypeStruct((qloc, 3), f32),
    )
    scratch = [
        pltpu.VMEM((_CH // 8, 8), i32),
        pltpu.VMEM((_CH,), i32),
        pltpu.VMEM((_CH, 16), f32),
        pltpu.VMEM((_CH, c), f32),
        pltpu.VMEM((_CH, cf), f32),
        pltpu.VMEM((_CH, 3), f32),
        pltpu.VMEM((_CH, 3), f32),
        pltpu.VMEM((_CH,), f32),
        pltpu.VMEM((_CH, 3), f32),
        pltpu.VMEM((_CH, 3), f32),
        pltpu.VMEM((12, _L), f32),
        pltpu.SemaphoreType.DMA,
        pltpu.SemaphoreType.DMA,
    ]
    fn = pl.kernel(
        _body, out_type=out_type, mesh=mesh, scratch_types=scratch,
        compiler_params=pltpu.CompilerParams(
            needs_layout_passes=False, use_tc_tiling_on_sc=False))
    oemb, ocol, odir, oconf, oloc = fn(packed, emb2d, idx8, locw3, cam_b)

    return (oemb.reshape(b, r, sr, k, cf),
            ocol.reshape(b, r, sr, k, 3),
            odir.reshape(b, r, sr, k, 3),
            oconf.reshape(b, r, sr, k, 1),
            oloc.reshape(b, r, sr, 3))


# speedup vs baseline: 1.0919x; 1.0052x over previous
import functools

import jax
import jax.numpy as jnp
from jax import lax
from jax.experimental import pallas as pl
from jax.experimental.pallas import tpu as pltpu
from jax.experimental.pallas import tpu_sc as plsc

_L = 16
_CH = 128
_G = _CH // _L


def _pers_from_lanes(x, y, z, cam):
    r00, r01, r02, r10, r11, r12, r20, r21, r22, c0, c1, c2 = cam
    sx = x - c0
    sy = y - c1
    sz = z - c2
    u0 = r00 * sx + r10 * sy + r20 * sz
    u1 = r01 * sx + r11 * sy + r21 * sz
    u2 = r02 * sx + r12 * sy + r22 * sz
    return u0 / u2, u1 / u2, u2


def _body(pk_hbm, emb_hbm, idx_hbm, locw_hbm, cam_hbm,
          oemb_hbm, ocol_hbm, odir_hbm, oconf_hbm, oloc_hbm,
          idx8_v, idx_v, pk_v, emb_v, oemb_v, ocol_v, odir_v, oconf_v,
          loc_v, oloc_v, cam_v, sem, osem):
    q_total = idx_hbm.shape[0] * 8
    qloc_total = locw_hbm.shape[0]
    nw = 32
    qw = q_total // nw
    qlw = qloc_total // nw
    wid = lax.axis_index("s") * 2 + lax.axis_index("c")

    pltpu.sync_copy(cam_hbm, cam_v)
    cam = tuple(cam_v[j, :] for j in range(12))
    iota = lax.iota(jnp.int32, _L)
    iota3 = iota * 3
    cols = [jnp.full((_L,), c, jnp.int32) for c in range(10)]

    def outs_of(sub, base):
        return (
            (oemb_v.at[sub], oemb_hbm.at[pl.ds(base, _CH)]),
            (ocol_v.at[sub], ocol_hbm.at[pl.ds(base, _CH)]),
            (odir_v.at[sub], odir_hbm.at[pl.ds(base, _CH)]),
            (oconf_v.at[sub], oconf_hbm.at[pl.ds(base, _CH)]),
        )

    def chunk2(it2, carry):
        for sub in range(2):
            it = it2 * 2 + sub
            base = wid * qw + it * _CH
            pltpu.sync_copy(idx_hbm.at[pl.ds(base // 8, _CH // 8)], idx8_v)
            for g in range(_G):
                e = iota + g * _L
                idx_v[pl.ds(g * _L, _L)] = plsc.load_gather(
                    idx8_v,
                    [lax.shift_right_logical(e, 3), lax.bitwise_and(e, 7)])
            d0 = pltpu.async_copy(emb_hbm.at[idx_v], emb_v, sem)
            d1 = pltpu.async_copy(pk_hbm.at[idx_v], pk_v, sem)
            d0.wait()
            d1.wait()

            @pl.when(it2 > 0)
            def _():
                for src, dst in outs_of(sub, base):
                    pltpu.make_async_copy(src, dst, osem).wait()

            oemb_s, ocol_s, odir_s, oconf_s = (
                oemb_v.at[sub], ocol_v.at[sub], odir_v.at[sub],
                oconf_v.at[sub])
            for g in range(_G):
                rows = iota + g * _L
                x = plsc.load_gather(pk_v, [rows, cols[0]])
                y = plsc.load_gather(pk_v, [rows, cols[1]])
                z = plsc.load_gather(pk_v, [rows, cols[2]])
                xp, yp, zc = _pers_from_lanes(x, y, z, cam)
                for c, val in ((0, x), (1, y), (2, z), (3, xp), (4, yp),
                               (5, zc)):
                    plsc.store_scatter(oemb_s, [rows, cols[c]], val)
                for c in range(3):
                    plsc.store_scatter(
                        ocol_s, [rows, cols[c]],
                        plsc.load_gather(pk_v, [rows, cols[3 + c]]))
                for c in range(3):
                    plsc.store_scatter(
                        odir_s, [rows, cols[c]],
                        plsc.load_gather(pk_v, [rows, cols[6 + c]]))
                plsc.store_scatter(
                    oconf_s, [rows], plsc.load_gather(pk_v, [rows, cols[9]]))
            for p in range(_CH):
                oemb_s[p, pl.ds(6, 16)] = emb_v[p, pl.ds(0, 16)]
                oemb_s[p, pl.ds(22, 16)] = emb_v[p, pl.ds(16, 16)]
            for src, dst in outs_of(sub, base):
                pltpu.async_copy(src, dst, osem)
        return carry

    nit2 = qw // _CH // 2
    lax.fori_loop(0, nit2, chunk2, 0)
    lastbase = wid * qw + (2 * nit2 - 2) * _CH
    for sub in range(2):
        for src, dst in outs_of(sub, lastbase + sub * _CH):
            pltpu.make_async_copy(src, dst, osem).wait()

    def loc_chunk(it, carry):
        base = wid * qlw + it * _CH
        pltpu.sync_copy(locw_hbm.at[pl.ds(base, _CH)], loc_v)
        for g in range(_G):
            rows = iota + g * _L
            x = plsc.load_gather(loc_v, [rows, cols[0]])
            y = plsc.load_gather(loc_v, [rows, cols[1]])
            z = plsc.load_gather(loc_v, [rows, cols[2]])
            xp, yp, zc = _pers_from_lanes(x, y, z, cam)
            plsc.store_scatter(oloc_v, [rows, cols[0]], xp)
            plsc.store_scatter(oloc_v, [rows, cols[1]], yp)
            plsc.store_scatter(oloc_v, [rows, cols[2]], zc)
        pltpu.sync_copy(oloc_v, oloc_hbm.at[pl.ds(base, _CH)])
        return carry

    lax.fori_loop(0, qlw // _CH, loc_chunk, 0)


def kernel(points_xyz, points_embeding, points_conf, points_dir, points_color,
           sample_pidx, sample_loc_w, cam_rot, cam_pos):
    n = points_xyz.shape[0]
    b, r, sr, k = sample_pidx.shape
    c = points_embeding.shape[-1]
    q = b * r * sr * k
    qloc = b * r * sr
    cf = c + 6

    packed = jnp.concatenate(
        [points_xyz, points_color[0], points_dir[0], points_conf[0],
         jnp.zeros((n, 6), jnp.float32)], axis=-1)
    emb2d = points_embeding.reshape(n, c)
    idx8 = sample_pidx.reshape(q // 8, 8)
    locw3 = sample_loc_w.reshape(qloc, 3)
    cam12 = jnp.concatenate(
        [cam_rot.reshape(9), cam_pos.reshape(3)]).astype(jnp.float32)
    cam_b = jnp.tile(cam12[:, None], (1, _L))

    mesh = plsc.VectorSubcoreMesh(
        core_axis_name="c", subcore_axis_name="s", num_cores=2, num_subcores=16)
    f32 = jnp.float32
    i32 = jnp.int32
    out_type = (
        jax.ShapeDtypeStruct((q, cf), f32),
        jax.ShapeDtypeStruct((q, 3), f32),
        jax.ShapeDtypeStruct((q, 3), f32),
        jax.ShapeDtypeStruct((q,), f32),
        jax.ShapeDtypeStruct((qloc, 3), f32),
    )
    scratch = [
        pltpu.VMEM((_CH // 8, 8), i32),
        pltpu.VMEM((_CH,), i32),
        pltpu.VMEM((_CH, 16), f32),
        pltpu.VMEM((_CH, c), f32),
        pltpu.VMEM((2, _CH, cf), f32),
        pltpu.VMEM((2, _CH, 3), f32),
        pltpu.VMEM((2, _CH, 3), f32),
        pltpu.VMEM((2, _CH), f32),
        pltpu.VMEM((_CH, 3), f32),
        pltpu.VMEM((_CH, 3), f32),
        pltpu.VMEM((12, _L), f32),
        pltpu.SemaphoreType.DMA,
        pltpu.SemaphoreType.DMA,
    ]
    fn = pl.kernel(
        _body, out_type=out_type, mesh=mesh, scratch_types=scratch,
        compiler_params=pltpu.CompilerParams(
            needs_layout_passes=False, use_tc_tiling_on_sc=False))
    oemb, ocol, odir, oconf, oloc = fn(packed, emb2d, idx8, locw3, cam_b)

    return (oemb.reshape(b, r, sr, k, cf),
            ocol.reshape(b, r, sr, k, 3),
            odir.reshape(b, r, sr, k, 3),
            oconf.reshape(b, r, sr, k, 1),
            oloc.reshape(b, r, sr, 3))
